# baseline (speedup 1.0000x reference)
import functools

import jax
import jax.numpy as jnp
from jax import lax
from jax.experimental import pallas as pl
from jax.experimental.pallas import tpu as pltpu
from jax.experimental.pallas import tpu_sc as plsc

N = 10000
E = 320000
F_IN = 128
HID = 64
HID2 = 32
N_DCS = 32
N_GRAPHS = 16

NC = 2
NS = 16
NW = NC * NS
CH = 128
K = 80
E_PAD = NW * K * CH
NP = 10112
RPT = NP // NS

_mesh = plsc.VectorSubcoreMesh(core_axis_name="c", subcore_axis_name="s")


NBG = 2
KA = NW * K // NS
NGA = KA // (2 * NBG)


def _make_agg(width):
  @functools.partial(
      pl.kernel,
      out_type=jax.ShapeDtypeStruct((NP, width), jnp.float32),
      mesh=_mesh,
      compiler_params=pltpu.CompilerParams(use_tc_tiling_on_sc=False),
      scratch_types=[
          pltpu.VMEM((KA, CH), jnp.int32),
          pltpu.VMEM((KA, CH), jnp.int32),
          pltpu.VMEM((2, NBG, CH, width), jnp.float32),
          pltpu.VMEM_SHARED((NP, width), jnp.float32),
          pltpu.SemaphoreType.DMA((2, NBG)),
      ],
  )
  def agg(src_hbm, dst_hbm, table_hbm, zeros_hbm, out_hbm,
          idx_s, idx_d, rows, acc, gsem):
    c = lax.axis_index("c")
    s = lax.axis_index("s")

    @pl.when(c == 0)
    def _work():
      pltpu.sync_copy(zeros_hbm, acc.at[pl.ds(s * RPT, RPT)])
      pltpu.sync_copy(src_hbm.at[pl.ds(s * KA, KA)], idx_s)
      pltpu.sync_copy(dst_hbm.at[pl.ds(s * KA, KA)], idx_d)
      plsc.subcore_barrier()

      def fire(p, grp):
        for b in range(NBG):
          jj = grp * NBG + b
          pltpu.async_copy(table_hbm.at[idx_s.at[jj]], rows.at[p, b],
                           gsem.at[p, b])

      def drain(p, grp):
        for b in range(NBG):
          jj = grp * NBG + b
          pltpu.make_async_copy(table_hbm.at[idx_s.at[jj]], rows.at[p, b],
                                gsem.at[p, b]).wait()
          pltpu.sync_copy(rows.at[p, b], acc.at[idx_d.at[jj]], add=True)

      fire(0, 0)

      def body(i, carry):
        fire(1, 2 * i + 1)
        drain(0, 2 * i)

        @pl.when(i + 1 < NGA)
        def _():
          fire(0, 2 * i + 2)

        drain(1, 2 * i + 1)
        return carry

      lax.fori_loop(0, NGA, body, 0)
      plsc.subcore_barrier()
      pltpu.sync_copy(acc.at[pl.ds(s * RPT, RPT)],
                      out_hbm.at[pl.ds(s * RPT, RPT)])

  return agg


_agg64 = _make_agg(HID)
_agg32 = _make_agg(HID2)

DEGW = 16


@functools.partial(
    pl.kernel,
    out_type=jax.ShapeDtypeStruct((NC, NP, DEGW), jnp.float32),
    mesh=_mesh,
    compiler_params=pltpu.CompilerParams(use_tc_tiling_on_sc=False),
    scratch_types=[
        pltpu.VMEM((K, CH), jnp.int32),
        pltpu.VMEM((CH, DEGW), jnp.float32),
        pltpu.VMEM_SHARED((NP, DEGW), jnp.float32),
        pltpu.SemaphoreType.DMA,
    ],
)
def _deg(dst_hbm, zeros_hbm, ones_hbm, out_hbm, idx_d, rows, acc, sem):
  c = lax.axis_index("c")
  s = lax.axis_index("s")
  wid = c * NS + s
  pltpu.sync_copy(zeros_hbm, acc.at[pl.ds(s * RPT, RPT)])
  pltpu.sync_copy(dst_hbm.at[pl.ds(wid * K, K)], idx_d)
  pltpu.sync_copy(ones_hbm, rows)
  plsc.subcore_barrier()

  def body(j, carry):
    pltpu.sync_copy(rows, acc.at[idx_d.at[j]], add=True)
    return carry

  lax.fori_loop(0, K, body, 0)
  plsc.subcore_barrier()
  pltpu.sync_copy(acc.at[pl.ds(s * RPT, RPT)],
                  out_hbm.at[c, pl.ds(s * RPT, RPT)])


def _tc_pre_body(degp_ref, x_ref, w1_ref, g1_ref, dis_ref):
  deg = degp_ref[0, :N, 0:1] + degp_ref[1, :N, 0:1] + 1.0
  dis = lax.rsqrt(deg)
  h = jnp.dot(x_ref[...], w1_ref[...], preferred_element_type=jnp.float32)
  g1_ref[...] = h * dis
  dis_ref[...] = dis


def _tc_pre(degp, x, w1):
  return pl.pallas_call(
      _tc_pre_body,
      out_shape=(jax.ShapeDtypeStruct((N, HID), jnp.float32),
                 jax.ShapeDtypeStruct((N, 1), jnp.float32)),
  )(degp, x, w1)


def _tc_mid_body(p_ref, g1_ref, dis_ref, w2_ref, b1_ref, g2_ref):
  dis = dis_ref[...]
  acc = p_ref[:N, :] + g1_ref[...]
  bx = jnp.maximum(acc * dis + b1_ref[...], 0.0)
  g2_ref[...] = jnp.dot(bx, w2_ref[...],
                        preferred_element_type=jnp.float32) * dis


def _tc_mid(p, g1, dis, w2, b1r):
  return pl.pallas_call(
      _tc_mid_body,
      out_shape=jax.ShapeDtypeStruct((N, HID2), jnp.float32),
  )(p, g1, dis, w2, b1r)


def _tc_post_body(q_ref, g2_ref, dis_ref, b2_ref, batch_ref, wm_ref, bm_ref,
                  out_ref):
  acc = q_ref[:N, :] + g2_ref[...]
  cx = jnp.maximum(acc * dis_ref[...] + b2_ref[...], 0.0)
  b = batch_ref[...]
  neg = jnp.float32(-jnp.inf)
  cols = []
  for g in range(N_GRAPHS):
    m = (b == g)
    cols.append(jnp.max(jnp.where(m, cx, neg), axis=0, keepdims=True))
  px = jnp.concatenate(cols, axis=0)
  out_ref[...] = jnp.dot(px, wm_ref[...],
                         preferred_element_type=jnp.float32) + bm_ref[...]


def _tc_post(q, g2, dis, b2r, batch2d, wm, bmr):
  return pl.pallas_call(
      _tc_post_body,
      out_shape=jax.ShapeDtypeStruct((N_GRAPHS, N_DCS), jnp.float32),
  )(q, g2, dis, b2r, batch2d, wm, bmr)


@jax.jit
def kernel(x, edge_index, batch, W1, b1, W2, b2, Wm, bm):
  pad = E_PAD - E
  srcp = jnp.concatenate(
      [edge_index[0], jnp.full((pad,), N, jnp.int32)]).reshape(NW * K, CH)
  dstp = jnp.concatenate(
      [edge_index[1], jnp.full((pad,), N, jnp.int32)]).reshape(NW * K, CH)

  z16 = jnp.zeros((RPT, DEGW), jnp.float32)
  z64 = jnp.zeros((RPT, HID), jnp.float32)
  z32 = jnp.zeros((RPT, HID2), jnp.float32)
  ones16 = jnp.ones((CH, DEGW), jnp.float32)

  degp = _deg(dstp, z16, ones16)
  g1, dis = _tc_pre(degp, x, W1)
  g1p = jnp.pad(g1, ((0, NP - N), (0, 0)))
  p = _agg64(srcp, dstp, g1p, z64)
  g2 = _tc_mid(p, g1, dis, W2, b1.reshape(1, HID))
  g2p = jnp.pad(g2, ((0, NP - N), (0, 0)))
  q = _agg32(srcp, dstp, g2p, z32)
  return _tc_post(q, g2, dis, b2.reshape(1, HID2),
                  batch.reshape(N, 1), Wm, bm.reshape(1, N_DCS))


# speedup vs baseline: 1.0711x; 1.0711x over previous
import functools

import jax
import jax.numpy as jnp
from jax import lax
from jax.experimental import pallas as pl
from jax.experimental.pallas import tpu as pltpu
from jax.experimental.pallas import tpu_sc as plsc

N = 10000
E = 320000
F_IN = 128
HID = 64
HID2 = 32
N_DCS = 32
N_GRAPHS = 16

NC = 2
NS = 16
NW = NC * NS
CH = 128
K = 80
E_PAD = NW * K * CH
NP = 10112
RPT = NP // NS

_mesh = plsc.VectorSubcoreMesh(core_axis_name="c", subcore_axis_name="s")


NBG = 4
NG = K // (2 * NBG)


def _make_agg(width):
  @functools.partial(
      pl.kernel,
      out_type=jax.ShapeDtypeStruct((NC, NP, width), jnp.float32),
      mesh=_mesh,
      compiler_params=pltpu.CompilerParams(use_tc_tiling_on_sc=False),
      scratch_types=[
          pltpu.VMEM((K, CH), jnp.int32),
          pltpu.VMEM((K, CH), jnp.int32),
          pltpu.VMEM((2, NBG, CH, width), jnp.float32),
          pltpu.VMEM_SHARED((NP, width), jnp.float32),
          pltpu.SemaphoreType.DMA((2, NBG)),
      ],
  )
  def agg(src_hbm, dst_hbm, table0_hbm, table1_hbm, zeros_hbm, out_hbm,
          idx_s, idx_d, rows, acc, gsem):
    c = lax.axis_index("c")
    s = lax.axis_index("s")
    wid = c * NS + s
    pltpu.sync_copy(zeros_hbm, acc.at[pl.ds(s * RPT, RPT)])
    pltpu.sync_copy(src_hbm.at[pl.ds(wid * K, K)], idx_s)
    pltpu.sync_copy(dst_hbm.at[pl.ds(wid * K, K)], idx_d)
    plsc.subcore_barrier()

    def run_core(table_hbm):
      def fire(p, grp):
        for b in range(NBG):
          jj = grp * NBG + b
          pltpu.async_copy(table_hbm.at[idx_s.at[jj]], rows.at[p, b],
                           gsem.at[p, b])

      def drain(p, grp):
        for b in range(NBG):
          jj = grp * NBG + b
          pltpu.make_async_copy(table_hbm.at[idx_s.at[jj]], rows.at[p, b],
                                gsem.at[p, b]).wait()
          pltpu.sync_copy(rows.at[p, b], acc.at[idx_d.at[jj]], add=True)

      fire(0, 0)

      def body(i, carry):
        fire(1, 2 * i + 1)
        drain(0, 2 * i)

        @pl.when(i + 1 < NG)
        def _():
          fire(0, 2 * i + 2)

        drain(1, 2 * i + 1)
        return carry

      lax.fori_loop(0, NG, body, 0)

    @pl.when(c == 0)
    def _():
      run_core(table0_hbm)

    @pl.when(c == 1)
    def _():
      run_core(table1_hbm)

    plsc.subcore_barrier()
    pltpu.sync_copy(acc.at[pl.ds(s * RPT, RPT)],
                    out_hbm.at[c, pl.ds(s * RPT, RPT)])

  return agg


_agg64 = _make_agg(HID)
_agg32 = _make_agg(HID2)

DEGW = 16


@functools.partial(
    pl.kernel,
    out_type=jax.ShapeDtypeStruct((NC, NP, DEGW), jnp.float32),
    mesh=_mesh,
    compiler_params=pltpu.CompilerParams(use_tc_tiling_on_sc=False),
    scratch_types=[
        pltpu.VMEM((K, CH), jnp.int32),
        pltpu.VMEM((CH, DEGW), jnp.float32),
        pltpu.VMEM_SHARED((NP, DEGW), jnp.float32),
        pltpu.SemaphoreType.DMA,
    ],
)
def _deg(dst_hbm, zeros_hbm, ones_hbm, out_hbm, idx_d, rows, acc, sem):
  c = lax.axis_index("c")
  s = lax.axis_index("s")
  wid = c * NS + s
  pltpu.sync_copy(zeros_hbm, acc.at[pl.ds(s * RPT, RPT)])
  pltpu.sync_copy(dst_hbm.at[pl.ds(wid * K, K)], idx_d)
  pltpu.sync_copy(ones_hbm, rows)
  plsc.subcore_barrier()

  def body(j, carry):
    pltpu.sync_copy(rows, acc.at[idx_d.at[j]], add=True)
    return carry

  lax.fori_loop(0, K, body, 0)
  plsc.subcore_barrier()
  pltpu.sync_copy(acc.at[pl.ds(s * RPT, RPT)],
                  out_hbm.at[c, pl.ds(s * RPT, RPT)])


def _tc_pre_body(degp_ref, x_ref, w1_ref, g1_ref, dis_ref):
  deg = degp_ref[0, :N, 0:1] + degp_ref[1, :N, 0:1] + 1.0
  dis = lax.rsqrt(deg)
  h = jnp.dot(x_ref[...], w1_ref[...], preferred_element_type=jnp.float32)
  g1_ref[...] = h * dis
  dis_ref[...] = dis


def _tc_pre(degp, x, w1):
  return pl.pallas_call(
      _tc_pre_body,
      out_shape=(jax.ShapeDtypeStruct((N, HID), jnp.float32),
                 jax.ShapeDtypeStruct((N, 1), jnp.float32)),
  )(degp, x, w1)


def _tc_mid_body(p_ref, g1_ref, dis_ref, w2_ref, b1_ref, g2_ref):
  dis = dis_ref[...]
  acc = p_ref[0, :N, :] + p_ref[1, :N, :] + g1_ref[...]
  bx = jnp.maximum(acc * dis + b1_ref[...], 0.0)
  g2_ref[...] = jnp.dot(bx, w2_ref[...],
                        preferred_element_type=jnp.float32) * dis


def _tc_mid(p, g1, dis, w2, b1r):
  return pl.pallas_call(
      _tc_mid_body,
      out_shape=jax.ShapeDtypeStruct((N, HID2), jnp.float32),
  )(p, g1, dis, w2, b1r)


def _tc_post_body(q_ref, g2_ref, dis_ref, b2_ref, batch_ref, wm_ref, bm_ref,
                  out_ref):
  acc = q_ref[0, :N, :] + q_ref[1, :N, :] + g2_ref[...]
  cx = jnp.maximum(acc * dis_ref[...] + b2_ref[...], 0.0)
  b = batch_ref[...]
  neg = jnp.float32(-jnp.inf)
  cols = []
  for g in range(N_GRAPHS):
    m = (b == g)
    cols.append(jnp.max(jnp.where(m, cx, neg), axis=0, keepdims=True))
  px = jnp.concatenate(cols, axis=0)
  out_ref[...] = jnp.dot(px, wm_ref[...],
                         preferred_element_type=jnp.float32) + bm_ref[...]


def _tc_post(q, g2, dis, b2r, batch2d, wm, bmr):
  return pl.pallas_call(
      _tc_post_body,
      out_shape=jax.ShapeDtypeStruct((N_GRAPHS, N_DCS), jnp.float32),
  )(q, g2, dis, b2r, batch2d, wm, bmr)


@jax.jit
def kernel(x, edge_index, batch, W1, b1, W2, b2, Wm, bm):
  pad = E_PAD - E
  srcp = jnp.concatenate(
      [edge_index[0], jnp.full((pad,), N, jnp.int32)]).reshape(NW * K, CH)
  dstp = jnp.concatenate(
      [edge_index[1], jnp.full((pad,), N, jnp.int32)]).reshape(NW * K, CH)

  z16 = jnp.zeros((RPT, DEGW), jnp.float32)
  z64 = jnp.zeros((RPT, HID), jnp.float32)
  z32 = jnp.zeros((RPT, HID2), jnp.float32)
  ones16 = jnp.ones((CH, DEGW), jnp.float32)

  degp = _deg(dstp, z16, ones16)
  g1, dis = _tc_pre(degp, x, W1)
  g1p = jnp.pad(g1, ((0, NP - N), (0, 0)))
  g1q = jnp.pad(g1, ((0, NP - N), (0, 0)), constant_values=1.0)
  p = _agg64(srcp, dstp, g1p, g1q, z64)
  g2 = _tc_mid(p, g1, dis, W2, b1.reshape(1, HID))
  g2p = jnp.pad(g2, ((0, NP - N), (0, 0)))
  g2q = jnp.pad(g2, ((0, NP - N), (0, 0)), constant_values=1.0)
  q = _agg32(srcp, dstp, g2p, g2q, z32)
  return _tc_post(q, g2, dis, b2.reshape(1, HID2),
                  batch.reshape(N, 1), Wm, bm.reshape(1, N_DCS))


# speedup vs baseline: 2.5522x; 2.3828x over previous
import functools

import jax
import jax.numpy as jnp
from jax import lax
from jax.experimental import pallas as pl
from jax.experimental.pallas import tpu as pltpu
from jax.experimental.pallas import tpu_sc as plsc

N = 10000
E = 320000
F_IN = 128
HID = 64
HID2 = 32
N_DCS = 32
N_GRAPHS = 16

NC = 2
NS = 16
NW = NC * NS
CH = 128
K = 80
E_PAD = NW * K * CH
NP = 10112
RPT = NP // NS

_mesh = plsc.VectorSubcoreMesh(core_axis_name="c", subcore_axis_name="s")


NBG = 4
NG = K // (2 * NBG)


def _make_agg(width):
  @functools.partial(
      pl.kernel,
      out_type=jax.ShapeDtypeStruct((NC, NP, width), jnp.float32),
      mesh=_mesh,
      compiler_params=pltpu.CompilerParams(use_tc_tiling_on_sc=False),
      scratch_types=[
          pltpu.VMEM((K, CH), jnp.int32),
          pltpu.VMEM((K, CH), jnp.int32),
          pltpu.VMEM((2, NBG, CH, width), jnp.float32),
          pltpu.VMEM_SHARED((NP, width), jnp.float32),
          pltpu.SemaphoreType.DMA((2, NBG)),
      ],
  )
  def agg(src_hbm, dst_hbm, table_hbm, zeros_hbm, out_hbm,
          idx_s, idx_d, rows, acc, gsem):
    c = lax.axis_index("c")
    s = lax.axis_index("s")
    wid = c * NS + s
    pltpu.sync_copy(zeros_hbm, acc.at[pl.ds(s * RPT, RPT)])
    pltpu.sync_copy(src_hbm.at[pl.ds(wid * K, K)], idx_s)
    pltpu.sync_copy(dst_hbm.at[pl.ds(wid * K, K)], idx_d)
    plsc.subcore_barrier()

    def fire(p, grp):
      for b in range(NBG):
        jj = grp * NBG + b
        pltpu.async_copy(table_hbm.at[idx_s.at[jj]], rows.at[p, b],
                         gsem.at[p, b])

    def drain(p, grp):
      for b in range(NBG):
        jj = grp * NBG + b
        pltpu.make_async_copy(table_hbm.at[idx_s.at[jj]], rows.at[p, b],
                              gsem.at[p, b]).wait()
        pltpu.sync_copy(rows.at[p, b], acc.at[idx_d.at[jj]], add=True)

    fire(0, 0)

    def body(i, carry):
      fire(1, 2 * i + 1)
      drain(0, 2 * i)

      @pl.when(i + 1 < NG)
      def _():
        fire(0, 2 * i + 2)

      drain(1, 2 * i + 1)
      return carry

    lax.fori_loop(0, NG, body, 0)
    plsc.subcore_barrier()
    pltpu.sync_copy(acc.at[pl.ds(s * RPT, RPT)],
                    out_hbm.at[c, pl.ds(s * RPT, RPT)])

  return agg


_agg64 = _make_agg(HID)
_agg32 = _make_agg(HID2)

DEGW = 16


@functools.partial(
    pl.kernel,
    out_type=jax.ShapeDtypeStruct((NC, NP, DEGW), jnp.float32),
    mesh=_mesh,
    compiler_params=pltpu.CompilerParams(use_tc_tiling_on_sc=False),
    scratch_types=[
        pltpu.VMEM((K, CH), jnp.int32),
        pltpu.VMEM((CH, DEGW), jnp.float32),
        pltpu.VMEM_SHARED((NP, DEGW), jnp.float32),
        pltpu.SemaphoreType.DMA,
    ],
)
def _deg(dst_hbm, zeros_hbm, ones_hbm, out_hbm, idx_d, rows, acc, sem):
  c = lax.axis_index("c")
  s = lax.axis_index("s")
  wid = c * NS + s
  pltpu.sync_copy(zeros_hbm, acc.at[pl.ds(s * RPT, RPT)])
  pltpu.sync_copy(dst_hbm.at[pl.ds(wid * K, K)], idx_d)
  pltpu.sync_copy(ones_hbm, rows)
  plsc.subcore_barrier()

  def body(j, carry):
    pltpu.sync_copy(rows, acc.at[idx_d.at[j]], add=True)
    return carry

  lax.fori_loop(0, K, body, 0)
  plsc.subcore_barrier()
  pltpu.sync_copy(acc.at[pl.ds(s * RPT, RPT)],
                  out_hbm.at[c, pl.ds(s * RPT, RPT)])


def _tc_pre_body(degp_ref, x_ref, w1_ref, g1_ref, dis_ref):
  deg = degp_ref[0, :N, 0:1] + degp_ref[1, :N, 0:1] + 1.0
  dis = lax.rsqrt(deg)
  h = jnp.dot(x_ref[...], w1_ref[...], preferred_element_type=jnp.float32)
  g1_ref[...] = h * dis
  dis_ref[...] = dis


def _tc_pre(degp, x, w1):
  return pl.pallas_call(
      _tc_pre_body,
      out_shape=(jax.ShapeDtypeStruct((N, HID), jnp.float32),
                 jax.ShapeDtypeStruct((N, 1), jnp.float32)),
  )(degp, x, w1)


def _tc_mid_body(p_ref, g1_ref, dis_ref, w2_ref, b1_ref, g2_ref):
  dis = dis_ref[...]
  acc = p_ref[0, :N, :] + p_ref[1, :N, :] + g1_ref[...]
  bx = jnp.maximum(acc * dis + b1_ref[...], 0.0)
  g2_ref[...] = jnp.dot(bx, w2_ref[...],
                        preferred_element_type=jnp.float32) * dis


def _tc_mid(p, g1, dis, w2, b1r):
  return pl.pallas_call(
      _tc_mid_body,
      out_shape=jax.ShapeDtypeStruct((N, HID2), jnp.float32),
  )(p, g1, dis, w2, b1r)


def _tc_post_body(q_ref, g2_ref, dis_ref, b2_ref, batch_ref, wm_ref, bm_ref,
                  out_ref):
  acc = q_ref[0, :N, :] + q_ref[1, :N, :] + g2_ref[...]
  cx = jnp.maximum(acc * dis_ref[...] + b2_ref[...], 0.0)
  b = batch_ref[...]
  neg = jnp.float32(-jnp.inf)
  cols = []
  for g in range(N_GRAPHS):
    m = (b == g)
    cols.append(jnp.max(jnp.where(m, cx, neg), axis=0, keepdims=True))
  px = jnp.concatenate(cols, axis=0)
  out_ref[...] = jnp.dot(px, wm_ref[...],
                         preferred_element_type=jnp.float32) + bm_ref[...]


def _tc_post(q, g2, dis, b2r, batch2d, wm, bmr):
  return pl.pallas_call(
      _tc_post_body,
      out_shape=jax.ShapeDtypeStruct((N_GRAPHS, N_DCS), jnp.float32),
  )(q, g2, dis, b2r, batch2d, wm, bmr)


@jax.jit
def kernel(x, edge_index, batch, W1, b1, W2, b2, Wm, bm):
  pad = E_PAD - E
  pad_idx = N + (jnp.arange(pad, dtype=jnp.int32) % (NP - N))
  srcp = jnp.concatenate([edge_index[0], pad_idx]).reshape(NW * K, CH)
  dstp = jnp.concatenate([edge_index[1], pad_idx]).reshape(NW * K, CH)

  z16 = jnp.zeros((RPT, DEGW), jnp.float32)
  z64 = jnp.zeros((RPT, HID), jnp.float32)
  z32 = jnp.zeros((RPT, HID2), jnp.float32)
  ones16 = jnp.ones((CH, DEGW), jnp.float32)

  degp = _deg(dstp, z16, ones16)
  g1, dis = _tc_pre(degp, x, W1)
  g1p = jnp.pad(g1, ((0, NP - N), (0, 0)))
  p = _agg64(srcp, dstp, g1p, z64)
  g2 = _tc_mid(p, g1, dis, W2, b1.reshape(1, HID))
  g2p = jnp.pad(g2, ((0, NP - N), (0, 0)))
  q = _agg32(srcp, dstp, g2p, z32)
  return _tc_post(q, g2, dis, b2.reshape(1, HID2),
                  batch.reshape(N, 1), Wm, bm.reshape(1, N_DCS))


# speedup vs baseline: 2.6640x; 1.0438x over previous
import functools

import jax
import jax.numpy as jnp
import numpy as np
from jax import lax
from jax.experimental import pallas as pl
from jax.experimental.pallas import tpu as pltpu
from jax.experimental.pallas import tpu_sc as plsc

N = 10000
E = 320000
F_IN = 128
HID = 64
HID2 = 32
N_DCS = 32
N_GRAPHS = 16

NC = 2
NS = 16
NW = NC * NS
CH = 128
K = 80
E_PAD = NW * K * CH
NP = 10112
RPT = NP // NS

_mesh = plsc.VectorSubcoreMesh(core_axis_name="c", subcore_axis_name="s")


NBG = 4
NG = K // (2 * NBG)


def _make_agg(width):
  @functools.partial(
      pl.kernel,
      out_type=jax.ShapeDtypeStruct((NC, NP, width), jnp.float32),
      mesh=_mesh,
      compiler_params=pltpu.CompilerParams(use_tc_tiling_on_sc=False),
      scratch_types=[
          pltpu.VMEM((K, CH), jnp.int32),
          pltpu.VMEM((K, CH), jnp.int32),
          pltpu.VMEM((2, NBG, CH, width), jnp.float32),
          pltpu.VMEM_SHARED((NP, width), jnp.float32),
          pltpu.SemaphoreType.DMA((2, NBG)),
      ],
  )
  def agg(src_hbm, dst_hbm, table_hbm, zeros_hbm, out_hbm,
          idx_s, idx_d, rows, acc, gsem):
    c = lax.axis_index("c")
    s = lax.axis_index("s")
    wid = c * NS + s
    pltpu.sync_copy(zeros_hbm, acc.at[pl.ds(s * RPT, RPT)])
    pltpu.sync_copy(src_hbm.at[pl.ds(wid * K, K)], idx_s)
    pltpu.sync_copy(dst_hbm.at[pl.ds(wid * K, K)], idx_d)
    plsc.subcore_barrier()

    def fire(p, grp):
      for b in range(NBG):
        jj = grp * NBG + b
        pltpu.async_copy(table_hbm.at[idx_s.at[jj]], rows.at[p, b],
                         gsem.at[p, b])

    def drain(p, grp):
      for b in range(NBG):
        jj = grp * NBG + b
        pltpu.make_async_copy(table_hbm.at[idx_s.at[jj]], rows.at[p, b],
                              gsem.at[p, b]).wait()
        pltpu.sync_copy(rows.at[p, b], acc.at[idx_d.at[jj]], add=True)

    fire(0, 0)

    def body(i, carry):
      fire(1, 2 * i + 1)
      drain(0, 2 * i)

      @pl.when(i + 1 < NG)
      def _():
        fire(0, 2 * i + 2)

      drain(1, 2 * i + 1)
      return carry

    lax.fori_loop(0, NG, body, 0)
    plsc.subcore_barrier()
    pltpu.sync_copy(acc.at[pl.ds(s * RPT, RPT)],
                    out_hbm.at[c, pl.ds(s * RPT, RPT)])

  return agg


_agg64 = _make_agg(HID)
_agg32 = _make_agg(HID2)

DEGW = 16


@functools.partial(
    pl.kernel,
    out_type=jax.ShapeDtypeStruct((NC, NP, DEGW), jnp.float32),
    mesh=_mesh,
    scratch_types=[
        pltpu.VMEM((K, CH), jnp.int32),
        pltpu.VMEM((CH, DEGW), jnp.float32),
        pltpu.VMEM_SHARED((NP, DEGW), jnp.float32),
        pltpu.SemaphoreType.DMA,
    ],
)
def _deg(dst_hbm, zeros_hbm, ones_hbm, out_hbm, idx_d, rows, acc, sem):
  c = lax.axis_index("c")
  s = lax.axis_index("s")
  wid = c * NS + s
  pltpu.sync_copy(zeros_hbm, acc.at[pl.ds(s * RPT, RPT)])
  pltpu.sync_copy(dst_hbm.at[pl.ds(wid * K, K)], idx_d)
  pltpu.sync_copy(ones_hbm, rows)
  plsc.subcore_barrier()

  def body(j, carry):
    pltpu.sync_copy(rows, acc.at[idx_d.at[j]], add=True)
    return carry

  lax.fori_loop(0, K, body, 0)
  plsc.subcore_barrier()
  pltpu.sync_copy(acc.at[pl.ds(s * RPT, RPT)],
                  out_hbm.at[c, pl.ds(s * RPT, RPT)])


def _tc_pre_body(degp_ref, x_ref, w1_ref, g1_ref, dis_ref):
  deg = degp_ref[0, :N, 0:1] + degp_ref[1, :N, 0:1] + 1.0
  dis = lax.rsqrt(deg)
  h = jnp.dot(x_ref[...], w1_ref[...], preferred_element_type=jnp.float32)
  g1_ref[...] = h * dis
  dis_ref[...] = dis


def _tc_pre(degp, x, w1):
  return pl.pallas_call(
      _tc_pre_body,
      out_shape=(jax.ShapeDtypeStruct((N, HID), jnp.float32),
                 jax.ShapeDtypeStruct((N, 1), jnp.float32)),
  )(degp, x, w1)


def _tc_mid_body(p_ref, g1_ref, dis_ref, w2_ref, b1_ref, g2_ref):
  dis = dis_ref[...]
  acc = p_ref[0, :N, :] + p_ref[1, :N, :] + g1_ref[...]
  bx = jnp.maximum(acc * dis + b1_ref[...], 0.0)
  g2_ref[...] = jnp.dot(bx, w2_ref[...],
                        preferred_element_type=jnp.float32) * dis


def _tc_mid(p, g1, dis, w2, b1r):
  return pl.pallas_call(
      _tc_mid_body,
      out_shape=jax.ShapeDtypeStruct((N, HID2), jnp.float32),
  )(p, g1, dis, w2, b1r)


def _tc_post_body(q_ref, g2_ref, dis_ref, b2_ref, batch_ref, wm_ref, bm_ref,
                  out_ref):
  acc = q_ref[0, :, :N] + q_ref[1, :, :N] + g2_ref[...]
  cx = jnp.maximum(acc * dis_ref[...] + b2_ref[...], 0.0)
  b = batch_ref[...]
  neg = jnp.float32(-jnp.inf)
  cols = []
  for g in range(N_GRAPHS):
    m = (b == g)
    cols.append(jnp.max(jnp.where(m, cx, neg), axis=1, keepdims=False))
  px = jnp.stack(cols, axis=0)
  out_ref[...] = jnp.dot(px, wm_ref[...],
                         preferred_element_type=jnp.float32) + bm_ref[...]


def _tc_post(qt, g2t, dist, b2t, batcht, wm, bmr):
  return pl.pallas_call(
      _tc_post_body,
      out_shape=jax.ShapeDtypeStruct((N_GRAPHS, N_DCS), jnp.float32),
  )(qt, g2t, dist, b2t, batcht, wm, bmr)


@jax.jit
def kernel(x, edge_index, batch, W1, b1, W2, b2, Wm, bm):
  pad = E_PAD - E
  pad_idx = jnp.asarray(N + (np.arange(pad) % (NP - N)), dtype=jnp.int32)
  srcp = jnp.concatenate([edge_index[0], pad_idx]).reshape(NW * K, CH)
  dstp = jnp.concatenate([edge_index[1], pad_idx]).reshape(NW * K, CH)

  z16 = jnp.zeros((RPT, DEGW), jnp.float32)
  z64 = jnp.zeros((RPT, HID), jnp.float32)
  z32 = jnp.zeros((RPT, HID2), jnp.float32)
  ones16 = jnp.ones((CH, DEGW), jnp.float32)

  degp = _deg(dstp, z16, ones16)
  g1, dis = _tc_pre(degp, x, W1)
  g1p = jnp.pad(g1, ((0, NP - N), (0, 0)))
  p = _agg64(srcp, dstp, g1p, z64)
  g2 = _tc_mid(p, g1, dis, W2, b1.reshape(1, HID))
  g2p = jnp.pad(g2, ((0, NP - N), (0, 0)))
  q = _agg32(srcp, dstp, g2p, z32)
  qt = jnp.transpose(q, (0, 2, 1))
  return _tc_post(qt, g2.T, dis.reshape(1, N), b2.reshape(HID2, 1),
                  batch.reshape(1, N), Wm, bm.reshape(1, N_DCS))


# speedup vs baseline: 2.7089x; 1.0169x over previous
import functools

import jax
import jax.numpy as jnp
import numpy as np
from jax import lax
from jax.experimental import pallas as pl
from jax.experimental.pallas import tpu as pltpu
from jax.experimental.pallas import tpu_sc as plsc

N = 10000
E = 320000
F_IN = 128
HID = 64
HID2 = 32
N_DCS = 32
N_GRAPHS = 16

NC = 2
NS = 16
NW = NC * NS
CH = 128
K = 80
E_PAD = NW * K * CH
NP = 10112
RPT = NP // NS

_mesh = plsc.VectorSubcoreMesh(core_axis_name="c", subcore_axis_name="s")


NBG = 4
NG = K // (2 * NBG)


def _make_agg(width):
  @functools.partial(
      pl.kernel,
      out_type=jax.ShapeDtypeStruct((NC, NP, width), jnp.float32),
      mesh=_mesh,
      compiler_params=pltpu.CompilerParams(use_tc_tiling_on_sc=False),
      scratch_types=[
          pltpu.VMEM((K, CH), jnp.int32),
          pltpu.VMEM((K, CH), jnp.int32),
          pltpu.VMEM((2, NBG, CH, width), jnp.float32),
          pltpu.VMEM_SHARED((NP, width), jnp.float32),
          pltpu.SemaphoreType.DMA((2, NBG)),
      ],
  )
  def agg(src_hbm, dst_hbm, table_hbm, zeros_hbm, out_hbm,
          idx_s, idx_d, rows, acc, gsem):
    c = lax.axis_index("c")
    s = lax.axis_index("s")
    wid = c * NS + s
    pltpu.sync_copy(zeros_hbm, acc.at[pl.ds(s * RPT, RPT)])
    pltpu.sync_copy(src_hbm.at[pl.ds(wid * K, K)], idx_s)
    pltpu.sync_copy(dst_hbm.at[pl.ds(wid * K, K)], idx_d)
    plsc.subcore_barrier()

    def fire(p, grp):
      for b in range(NBG):
        jj = grp * NBG + b
        pltpu.async_copy(table_hbm.at[idx_s.at[jj]], rows.at[p, b],
                         gsem.at[p, b])

    def drain(p, grp):
      for b in range(NBG):
        jj = grp * NBG + b
        pltpu.make_async_copy(table_hbm.at[idx_s.at[jj]], rows.at[p, b],
                              gsem.at[p, b]).wait()
        pltpu.sync_copy(rows.at[p, b], acc.at[idx_d.at[jj]], add=True)

    fire(0, 0)

    def body(i, carry):
      fire(1, 2 * i + 1)
      drain(0, 2 * i)

      @pl.when(i + 1 < NG)
      def _():
        fire(0, 2 * i + 2)

      drain(1, 2 * i + 1)
      return carry

    lax.fori_loop(0, NG, body, 0)
    plsc.subcore_barrier()
    pltpu.sync_copy(acc.at[pl.ds(s * RPT, RPT)],
                    out_hbm.at[c, pl.ds(s * RPT, RPT)])

  return agg


_agg64 = _make_agg(HID)
_agg32 = _make_agg(HID2)

DEGW = 16


@functools.partial(
    pl.kernel,
    out_type=jax.ShapeDtypeStruct((NC, NP, DEGW), jnp.float32),
    mesh=_mesh,
    compiler_params=pltpu.CompilerParams(use_tc_tiling_on_sc=False),
    scratch_types=[
        pltpu.VMEM((K, CH), jnp.int32),
        pltpu.VMEM((CH, DEGW), jnp.float32),
        pltpu.VMEM_SHARED((NP, DEGW), jnp.float32),
        pltpu.SemaphoreType.DMA,
    ],
)
def _deg(dst_hbm, zeros_hbm, ones_hbm, out_hbm, idx_d, rows, acc, sem):
  c = lax.axis_index("c")
  s = lax.axis_index("s")
  wid = c * NS + s
  pltpu.sync_copy(zeros_hbm, acc.at[pl.ds(s * RPT, RPT)])
  pltpu.sync_copy(dst_hbm.at[pl.ds(wid * K, K)], idx_d)
  pltpu.sync_copy(ones_hbm, rows)
  plsc.subcore_barrier()

  def body(j, carry):
    pltpu.sync_copy(rows, acc.at[idx_d.at[j]], add=True)
    return carry

  lax.fori_loop(0, K, body, 0)
  plsc.subcore_barrier()
  pltpu.sync_copy(acc.at[pl.ds(s * RPT, RPT)],
                  out_hbm.at[c, pl.ds(s * RPT, RPT)])


def _tc_pre_body(degp_ref, x_ref, w1_ref, g1_ref, dis_ref):
  deg = degp_ref[0, :N, 0:1] + degp_ref[1, :N, 0:1] + 1.0
  dis = lax.rsqrt(deg)
  h = jnp.dot(x_ref[...], w1_ref[...], preferred_element_type=jnp.float32)
  g1_ref[...] = h * dis
  dis_ref[...] = dis


def _tc_pre(degp, x, w1):
  return pl.pallas_call(
      _tc_pre_body,
      out_shape=(jax.ShapeDtypeStruct((N, HID), jnp.float32),
                 jax.ShapeDtypeStruct((N, 1), jnp.float32)),
  )(degp, x, w1)


def _tc_mid_body(p_ref, g1_ref, dis_ref, w2_ref, b1_ref, g2_ref):
  dis = dis_ref[...]
  acc = p_ref[0, :N, :] + p_ref[1, :N, :] + g1_ref[...]
  bx = jnp.maximum(acc * dis + b1_ref[...], 0.0)
  g2_ref[...] = jnp.dot(bx, w2_ref[...],
                        preferred_element_type=jnp.float32) * dis


def _tc_mid(p, g1, dis, w2, b1r):
  return pl.pallas_call(
      _tc_mid_body,
      out_shape=jax.ShapeDtypeStruct((N, HID2), jnp.float32),
  )(p, g1, dis, w2, b1r)


def _tc_post_body(q_ref, g2_ref, dis_ref, b2_ref, batch_ref, wm_ref, bm_ref,
                  out_ref):
  acc = q_ref[0, :, :N] + q_ref[1, :, :N] + g2_ref[...]
  cx = jnp.maximum(acc * dis_ref[...] + b2_ref[...], 0.0)
  b = batch_ref[...]
  neg = jnp.float32(-jnp.inf)
  cols = []
  for g in range(N_GRAPHS):
    m = (b == g)
    cols.append(jnp.max(jnp.where(m, cx, neg), axis=1, keepdims=False))
  px = jnp.stack(cols, axis=0)
  out_ref[...] = jnp.dot(px, wm_ref[...],
                         preferred_element_type=jnp.float32) + bm_ref[...]


def _tc_post(qt, g2t, dist, b2t, batcht, wm, bmr):
  return pl.pallas_call(
      _tc_post_body,
      out_shape=jax.ShapeDtypeStruct((N_GRAPHS, N_DCS), jnp.float32),
  )(qt, g2t, dist, b2t, batcht, wm, bmr)


@jax.jit
def kernel(x, edge_index, batch, W1, b1, W2, b2, Wm, bm):
  pad = E_PAD - E
  pad_idx = jnp.asarray(N + (np.arange(pad) % (NP - N)), dtype=jnp.int32)
  srcp = jnp.concatenate([edge_index[0], pad_idx]).reshape(NW * K, CH)
  dstp = jnp.concatenate([edge_index[1], pad_idx]).reshape(NW * K, CH)

  z16 = jnp.zeros((RPT, DEGW), jnp.float32)
  z64 = jnp.zeros((RPT, HID), jnp.float32)
  z32 = jnp.zeros((RPT, HID2), jnp.float32)
  ones16 = jnp.ones((CH, DEGW), jnp.float32)

  degp = _deg(dstp, z16, ones16)
  g1, dis = _tc_pre(degp, x, W1)
  g1p = jnp.pad(g1, ((0, NP - N), (0, 0)))
  p = _agg64(srcp, dstp, g1p, z64)
  g2 = _tc_mid(p, g1, dis, W2, b1.reshape(1, HID))
  g2p = jnp.pad(g2, ((0, NP - N), (0, 0)))
  q = _agg32(srcp, dstp, g2p, z32)
  qt = jnp.transpose(q, (0, 2, 1))
  return _tc_post(qt, g2.T, dis.reshape(1, N), b2.reshape(HID2, 1),
                  batch.reshape(1, N), Wm, bm.reshape(1, N_DCS))


# speedup vs baseline: 2.7139x; 1.0018x over previous
import functools

import jax
import jax.numpy as jnp
import numpy as np
from jax import lax
from jax.experimental import pallas as pl
from jax.experimental.pallas import tpu as pltpu
from jax.experimental.pallas import tpu_sc as plsc

N = 10000
E = 320000
F_IN = 128
HID = 64
HID2 = 32
N_DCS = 32
N_GRAPHS = 16

NC = 2
NS = 16
NW = NC * NS
CH = 128
K = 80
E_PAD = NW * K * CH
NP = 10112
RPT = NP // NS

_mesh = plsc.VectorSubcoreMesh(core_axis_name="c", subcore_axis_name="s")


NBG = 4
NG = K // (2 * NBG)


def _make_agg(width):
  @functools.partial(
      pl.kernel,
      out_type=jax.ShapeDtypeStruct((NC, NP, width), jnp.float32),
      mesh=_mesh,
      compiler_params=pltpu.CompilerParams(use_tc_tiling_on_sc=False),
      scratch_types=[
          pltpu.VMEM((K, CH), jnp.int32),
          pltpu.VMEM((K, CH), jnp.int32),
          pltpu.VMEM((2, NBG, CH, width), jnp.float32),
          pltpu.VMEM_SHARED((NP, width), jnp.float32),
          pltpu.SemaphoreType.DMA((2, NBG)),
      ],
  )
  def agg(src_hbm, dst_hbm, table_hbm, zeros_hbm, out_hbm,
          idx_s, idx_d, rows, acc, gsem):
    c = lax.axis_index("c")
    s = lax.axis_index("s")
    wid = c * NS + s
    pltpu.sync_copy(zeros_hbm, acc.at[pl.ds(s * RPT, RPT)])
    pltpu.sync_copy(src_hbm.at[pl.ds(wid * K, K)], idx_s)
    pltpu.sync_copy(dst_hbm.at[pl.ds(wid * K, K)], idx_d)
    plsc.subcore_barrier()

    def fire(p, grp):
      for b in range(NBG):
        jj = grp * NBG + b
        pltpu.async_copy(table_hbm.at[idx_s.at[jj]], rows.at[p, b],
                         gsem.at[p, b])

    def drain(p, grp):
      for b in range(NBG):
        jj = grp * NBG + b
        pltpu.make_async_copy(table_hbm.at[idx_s.at[jj]], rows.at[p, b],
                              gsem.at[p, b]).wait()
        pltpu.sync_copy(rows.at[p, b], acc.at[idx_d.at[jj]], add=True)

    fire(0, 0)

    def body(i, carry):
      fire(1, 2 * i + 1)
      drain(0, 2 * i)

      @pl.when(i + 1 < NG)
      def _():
        fire(0, 2 * i + 2)

      drain(1, 2 * i + 1)
      return carry

    lax.fori_loop(0, NG, body, 0)
    plsc.subcore_barrier()
    pltpu.sync_copy(acc.at[pl.ds(s * RPT, RPT)],
                    out_hbm.at[c, pl.ds(s * RPT, RPT)])

  return agg


_agg64 = _make_agg(HID)
_agg32 = _make_agg(HID2)

DEGW = 16


@functools.partial(
    pl.kernel,
    out_type=jax.ShapeDtypeStruct((NC, NP, DEGW), jnp.float32),
    mesh=_mesh,
    compiler_params=pltpu.CompilerParams(use_tc_tiling_on_sc=False),
    scratch_types=[
        pltpu.VMEM((K, CH), jnp.int32),
        pltpu.VMEM((CH, DEGW), jnp.float32),
        pltpu.VMEM_SHARED((NP, DEGW), jnp.float32),
        pltpu.SemaphoreType.DMA,
    ],
)
def _deg(dst_hbm, zeros_hbm, ones_hbm, out_hbm, idx_d, rows, acc, sem):
  c = lax.axis_index("c")
  s = lax.axis_index("s")
  wid = c * NS + s
  pltpu.sync_copy(zeros_hbm, acc.at[pl.ds(s * RPT, RPT)])
  pltpu.sync_copy(dst_hbm.at[pl.ds(wid * K, K)], idx_d)
  pltpu.sync_copy(ones_hbm, rows)
  plsc.subcore_barrier()

  def body(j, carry):
    pltpu.sync_copy(rows, acc.at[idx_d.at[j]], add=True)
    return carry

  lax.fori_loop(0, K, body, 0)
  plsc.subcore_barrier()
  pltpu.sync_copy(acc.at[pl.ds(s * RPT, RPT)],
                  out_hbm.at[c, pl.ds(s * RPT, RPT)])


def _tc_pre_body(degp_ref, x_ref, w1_ref, g1_ref, dis_ref):
  deg = (degp_ref[0:1, :N] + degp_ref[1:2, :N] + 1.0).reshape(N, 1)
  dis = lax.rsqrt(deg)
  h = jnp.dot(x_ref[...], w1_ref[...], preferred_element_type=jnp.float32)
  g1_ref[...] = h * dis
  dis_ref[...] = dis


def _tc_pre(degp, x, w1):
  return pl.pallas_call(
      _tc_pre_body,
      out_shape=(jax.ShapeDtypeStruct((N, HID), jnp.float32),
                 jax.ShapeDtypeStruct((N, 1), jnp.float32)),
  )(degp, x, w1)


def _tc_mid_body(p_ref, g1_ref, dis_ref, w2_ref, b1_ref, g2_ref):
  dis = dis_ref[...]
  acc = p_ref[0, :N, :] + p_ref[1, :N, :] + g1_ref[...]
  bx = jnp.maximum(acc * dis + b1_ref[...], 0.0)
  g2_ref[...] = jnp.dot(bx, w2_ref[...],
                        preferred_element_type=jnp.float32) * dis


def _tc_mid(p, g1, dis, w2, b1r):
  return pl.pallas_call(
      _tc_mid_body,
      out_shape=jax.ShapeDtypeStruct((N, HID2), jnp.float32),
  )(p, g1, dis, w2, b1r)


def _tc_post_body(q_ref, g2_ref, dis_ref, b2_ref, batch_ref, wm_ref, bm_ref,
                  out_ref):
  acc = q_ref[0, :, :N] + q_ref[1, :, :N] + g2_ref[...]
  cx = jnp.maximum(acc * dis_ref[...] + b2_ref[...], 0.0)
  b = batch_ref[...]
  neg = jnp.float32(-jnp.inf)
  cols = []
  for g in range(N_GRAPHS):
    m = (b == g)
    cols.append(jnp.max(jnp.where(m, cx, neg), axis=1, keepdims=False))
  px = jnp.stack(cols, axis=0)
  out_ref[...] = jnp.dot(px, wm_ref[...],
                         preferred_element_type=jnp.float32) + bm_ref[...]


def _tc_post(qt, g2t, dist, b2t, batcht, wm, bmr):
  return pl.pallas_call(
      _tc_post_body,
      out_shape=jax.ShapeDtypeStruct((N_GRAPHS, N_DCS), jnp.float32),
  )(qt, g2t, dist, b2t, batcht, wm, bmr)


@jax.jit
def kernel(x, edge_index, batch, W1, b1, W2, b2, Wm, bm):
  pad = E_PAD - E
  pad_idx = jnp.asarray(N + (np.arange(pad) % (NP - N)), dtype=jnp.int32)
  ep = jnp.concatenate(
      [edge_index, jnp.broadcast_to(pad_idx, (2, pad))], axis=1)
  ep = ep.reshape(2, NW * K, CH)
  srcp, dstp = ep[0], ep[1]

  z16 = jnp.zeros((RPT, DEGW), jnp.float32)
  z64 = jnp.zeros((RPT, HID), jnp.float32)
  z32 = jnp.zeros((RPT, HID2), jnp.float32)
  ones16 = jnp.ones((CH, DEGW), jnp.float32)

  degp = _deg(dstp, z16, ones16)
  g1, dis = _tc_pre(degp[:, :, 0], x, W1)
  g1p = jnp.pad(g1, ((0, NP - N), (0, 0)))
  p = _agg64(srcp, dstp, g1p, z64)
  g2 = _tc_mid(p, g1, dis, W2, b1.reshape(1, HID))
  g2p = jnp.pad(g2, ((0, NP - N), (0, 0)))
  q = _agg32(srcp, dstp, g2p, z32)
  qt = jnp.transpose(q, (0, 2, 1))
  return _tc_post(qt, g2.T, dis.reshape(1, N), b2.reshape(HID2, 1),
                  batch.reshape(1, N), Wm, bm.reshape(1, N_DCS))
